# baseline (speedup 1.0000x reference)
import functools

import jax
import jax.numpy as jnp
from jax import lax
from jax.experimental import pallas as pl
from jax.experimental.pallas import tpu as pltpu
from jax.experimental.pallas import tpu_sc as plsc

N = 10000
E = 320000
D = 128
H = 128

NC = 2
NS = 16
NW = NC * NS
EC = 64
EPW = E // NS
CKT = -(-EPW // EC)
ECL = EPW - (CKT - 1) * EC
CK0 = 196
CK1 = CKT - CK0
NBUF = 5
NI = 8
LAI = 6
N_ACC = 10240
STRIPE = N_ACC // NS

_mesh = plsc.VectorSubcoreMesh(core_axis_name="c", subcore_axis_name="s")


@functools.partial(
    pl.kernel,
    out_type=jax.ShapeDtypeStruct((NC, N_ACC), jnp.float32),
    mesh=_mesh,
    compiler_params=pltpu.CompilerParams(needs_layout_passes=False),
    scratch_types=[
        pltpu.VMEM((N_ACC,), jnp.float32),
        pltpu.VMEM((CKT, EC), jnp.int32),
        pltpu.VMEM_SHARED((NS, N_ACC), jnp.float32),
        pltpu.VMEM((STRIPE,), jnp.float32),
        pltpu.VMEM((STRIPE,), jnp.float32),
    ],
)
def _deg_kernel(dst_hbm, out_hbm, hist, didx, shared, tmp, dacc):
    cid = lax.axis_index("c")
    sid = lax.axis_index("s")
    cbase = cid * CK0
    ckm = jnp.where(cid == 0, CK0, CK1)
    zeros16 = jnp.zeros((16,), jnp.float32)
    ones16 = jnp.ones((16,), jnp.float32)

    def _zero_hist(i, _):
        hist[pl.ds(i * 16, 16)] = zeros16
        return 0

    lax.fori_loop(0, N_ACC // 16, _zero_hist, 0)

    pltpu.sync_copy(dst_hbm.at[pl.ds(E + sid * EPW, EPW)],
                    didx.at[pl.ds(0, EPW)])
    trash16 = jnp.full((16,), N, jnp.int32)
    for j in range((CKT * EC - EPW) // 16):
        didx[pl.ds(EPW + j * 16, 16)] = trash16

    def _count(c, _):
        for j in range(EC // 16):
            idx = didx[pl.ds((cbase + c) * EC + j * 16, 16)]
            plsc.addupdate_scatter(hist, [idx], ones16)
        return 0

    lax.fori_loop(0, ckm, _count, 0)

    pltpu.sync_copy(hist, shared.at[sid])
    plsc.subcore_barrier()

    base = sid * STRIPE

    def _zero_acc(i, _):
        dacc[pl.ds(i * 16, 16)] = zeros16
        return 0

    lax.fori_loop(0, STRIPE // 16, _zero_acc, 0)

    for k in range(NS):
        pltpu.sync_copy(shared.at[k, pl.ds(base, STRIPE)], tmp)

        def _accum(i, _):
            s = pl.ds(i * 16, 16)
            dacc[s] = dacc[s] + tmp[s]
            return 0

        lax.fori_loop(0, STRIPE // 16, _accum, 0)

    pltpu.sync_copy(dacc, out_hbm.at[cid, pl.ds(base, STRIPE)])


@functools.partial(
    pl.kernel,
    out_type=jax.ShapeDtypeStruct((NC, N_ACC, H), jnp.float32),
    mesh=_mesh,
    compiler_params=pltpu.CompilerParams(needs_layout_passes=False),
    scratch_types=[
        pltpu.VMEM_SHARED((N_ACC, H), jnp.float32),
        pltpu.VMEM((NI, EC), jnp.int32),
        pltpu.VMEM((NI, EC), jnp.int32),
        pltpu.VMEM((NBUF, EC, H), jnp.float32),
        pltpu.SemaphoreType.DMA,
        pltpu.SemaphoreType.DMA,
        pltpu.SemaphoreType.DMA,
    ],
)
def _agg_kernel(hws_hbm, edge_hbm, zrows_hbm, out_hbm,
                acc, sidxr, didxr, rowsr, gsem, ssem, isem):
    cid = lax.axis_index("c")
    sid = lax.axis_index("s")
    cbase = cid * CK0
    ckm = jnp.where(cid == 0, CK0, CK1)
    ebase = sid * EPW
    row0 = sid * STRIPE

    pltpu.sync_copy(zrows_hbm, acc.at[pl.ds(row0, STRIPE)])
    for j in range(NBUF - 1):
        pltpu.sync_copy(edge_hbm.at[pl.ds(ebase + (cbase + j) * EC, EC)],
                        sidxr.at[j])
        pltpu.sync_copy(edge_hbm.at[pl.ds(E + ebase + (cbase + j) * EC, EC)],
                        didxr.at[j])
    for j in range(NBUF - 1, LAI):
        pltpu.async_copy(edge_hbm.at[pl.ds(ebase + (cbase + j) * EC, EC)],
                         sidxr.at[j], isem)
        pltpu.async_copy(edge_hbm.at[pl.ds(E + ebase + (cbase + j) * EC, EC)],
                         didxr.at[j], isem)
    plsc.subcore_barrier()

    EH = EC // 2
    for j in range(NBUF - 1):
        pltpu.async_copy(hws_hbm.at[sidxr.at[j, pl.ds(0, EH)]],
                         rowsr.at[j, pl.ds(0, EH)], gsem)
        pltpu.async_copy(hws_hbm.at[sidxr.at[j, pl.ds(EH, EH)]],
                         rowsr.at[j, pl.ds(EH, EH)], gsem)

    def _edges(c, _):
        b = c % NBUF
        s_c = c % NI
        pltpu.make_async_copy(
            hws_hbm.at[sidxr.at[s_c, pl.ds(0, EH)]],
            rowsr.at[b, pl.ds(0, EH)], gsem).wait()
        pltpu.make_async_copy(
            hws_hbm.at[sidxr.at[s_c, pl.ds(EH, EH)]],
            rowsr.at[b, pl.ds(EH, EH)], gsem).wait()
        pltpu.async_copy(rowsr.at[b], acc.at[didxr.at[s_c]], ssem, add=True)

        @pl.when(c > 0)
        def _wait_prev_scatter():
            bp = (c + NBUF - 1) % NBUF
            sp = (c + NI - 1) % NI
            pltpu.make_async_copy(
                rowsr.at[bp], acc.at[didxr.at[sp]], ssem).wait()

        @pl.when(c + NBUF - 1 < ckm)
        def _start_next_gather():
            bn = (c + NBUF - 1) % NBUF
            sn = (c + NBUF - 1) % NI
            g = cbase + c + NBUF - 1

            @pl.when(g < CKT - 1)
            def _wait_full_idx():
                pltpu.make_async_copy(
                    edge_hbm.at[pl.ds(ebase + g * EC, EC)],
                    sidxr.at[sn], isem).wait()
                pltpu.make_async_copy(
                    edge_hbm.at[pl.ds(E + ebase + g * EC, EC)],
                    didxr.at[sn], isem).wait()

            @pl.when(g == CKT - 1)
            def _wait_partial_idx():
                pltpu.make_async_copy(
                    edge_hbm.at[pl.ds(ebase + g * EC, ECL)],
                    sidxr.at[sn, pl.ds(0, ECL)], isem).wait()
                pltpu.make_async_copy(
                    edge_hbm.at[pl.ds(E + ebase + g * EC, ECL)],
                    didxr.at[sn, pl.ds(0, ECL)], isem).wait()

            pltpu.async_copy(hws_hbm.at[sidxr.at[sn, pl.ds(0, EH)]],
                             rowsr.at[bn, pl.ds(0, EH)], gsem)
            pltpu.async_copy(hws_hbm.at[sidxr.at[sn, pl.ds(EH, EH)]],
                             rowsr.at[bn, pl.ds(EH, EH)], gsem)

        @pl.when(c + LAI < ckm)
        def _start_next_idx_load():
            si = (c + LAI) % NI
            g = cbase + c + LAI

            @pl.when(g < CKT - 1)
            def _load_full_idx():
                pltpu.async_copy(
                    edge_hbm.at[pl.ds(ebase + g * EC, EC)],
                    sidxr.at[si], isem)
                pltpu.async_copy(
                    edge_hbm.at[pl.ds(E + ebase + g * EC, EC)],
                    didxr.at[si], isem)

            @pl.when(g == CKT - 1)
            def _load_partial_idx():
                pltpu.async_copy(
                    edge_hbm.at[pl.ds(ebase + g * EC, ECL)],
                    sidxr.at[si, pl.ds(0, ECL)], isem)
                pltpu.async_copy(
                    edge_hbm.at[pl.ds(E + ebase + g * EC, ECL)],
                    didxr.at[si, pl.ds(0, ECL)], isem)
                zero16 = jnp.zeros((16,), jnp.int32)
                trash16 = jnp.full((16,), N, jnp.int32)
                for j in range((EC - ECL) // 16):
                    sidxr[si, pl.ds(ECL + j * 16, 16)] = zero16
                    didxr[si, pl.ds(ECL + j * 16, 16)] = trash16

        return 0

    lax.fori_loop(0, ckm, _edges, 0)
    pltpu.make_async_copy(
        rowsr.at[(ckm - 1) % NBUF],
        acc.at[didxr.at[(ckm - 1) % NI]], ssem).wait()

    plsc.subcore_barrier()
    pltpu.sync_copy(acc.at[pl.ds(row0, STRIPE)],
                    out_hbm.at[cid, pl.ds(row0, STRIPE)])


def _mm_body(x_ref, w_ref, o_ref):
    o_ref[...] = jnp.dot(
        x_ref[...], w_ref[...], preferred_element_type=jnp.float32)


def _matmul(x, w):
    grid = 10
    rb = N // grid
    return pl.pallas_call(
        _mm_body,
        grid=(grid,),
        in_specs=[
            pl.BlockSpec((rb, D), lambda i: (i, 0)),
            pl.BlockSpec((D, H), lambda i: (0, 0)),
        ],
        out_specs=pl.BlockSpec((rb, H), lambda i: (i, 0)),
        out_shape=jax.ShapeDtypeStruct((N, H), jnp.float32),
    )(x, w)


def _bn_relu(t, g_ref, be_ref):
    mean = jnp.mean(t, axis=0, keepdims=True)
    var = jnp.mean((t - mean) ** 2, axis=0, keepdims=True)
    hn = (t - mean) * lax.rsqrt(var + 1e-5) * g_ref[...] + be_ref[...]
    return jnp.maximum(hn, 0.0)


def _post1_body(p_ref, hws_ref, u_ref, b_ref, g_ref, be_ref, w2_ref,
                h1_ref, hws2_ref):
    s = p_ref[0, :N, :] + p_ref[1, :N, :] + hws_ref[...]
    t = u_ref[...] * s + b_ref[...]
    h1 = _bn_relu(t, g_ref, be_ref)
    h1_ref[...] = h1
    hws2_ref[...] = u_ref[...] * jnp.dot(
        h1, w2_ref[...], preferred_element_type=jnp.float32)


def _post2_body(p_ref, hws_ref, u_ref, b_ref, g_ref, be_ref, h2_ref):
    s = p_ref[0, :N, :] + p_ref[1, :N, :] + hws_ref[...]
    t = u_ref[...] * s + b_ref[...]
    h2_ref[...] = _bn_relu(t, g_ref, be_ref)


def kernel(x, edge_index, W1, b1, gamma1, beta1, W2, b2, gamma2, beta2):
    eflat = edge_index.reshape(2 * E)
    deg2 = _deg_kernel(eflat)
    deg = deg2[0, :N] + deg2[1, :N] + 1.0
    u_col = lax.rsqrt(deg)[:, None]

    zrows = jnp.zeros((STRIPE, H), jnp.float32)
    b1r, g1r, be1r = b1[None, :], gamma1[None, :], beta1[None, :]
    b2r, g2r, be2r = b2[None, :], gamma2[None, :], beta2[None, :]

    hws1 = u_col * _matmul(x, W1)
    p1 = _agg_kernel(hws1, eflat, zrows)

    h1, hws2 = pl.pallas_call(
        _post1_body,
        out_shape=(jax.ShapeDtypeStruct((N, H), jnp.float32),
                   jax.ShapeDtypeStruct((N, H), jnp.float32)),
    )(p1, hws1, u_col, b1r, g1r, be1r, W2)

    p2 = _agg_kernel(hws2, eflat, zrows)

    h2 = pl.pallas_call(
        _post2_body,
        out_shape=jax.ShapeDtypeStruct((N, H), jnp.float32),
    )(p2, hws2, u_col, b2r, g2r, be2r)

    return (h1, h2)


# speedup vs baseline: 1.0203x; 1.0203x over previous
import functools

import jax
import jax.numpy as jnp
from jax import lax
from jax.experimental import pallas as pl
from jax.experimental.pallas import tpu as pltpu
from jax.experimental.pallas import tpu_sc as plsc

N = 10000
E = 320000
D = 128
H = 128

NC = 2
NS = 16
NW = NC * NS
EC = 64
EPW = E // NS
CKT = -(-EPW // EC)
ECL = EPW - (CKT - 1) * EC
CK0 = 196
CK1 = CKT - CK0
NBUF = 5
NI = 8
LAI = 6
N_ACC = 10240
STRIPE = N_ACC // NS

_mesh = plsc.VectorSubcoreMesh(core_axis_name="c", subcore_axis_name="s")


@functools.partial(
    pl.kernel,
    out_type=jax.ShapeDtypeStruct((NC, N_ACC), jnp.float32),
    mesh=_mesh,
    compiler_params=pltpu.CompilerParams(needs_layout_passes=False),
    scratch_types=[
        pltpu.VMEM((N_ACC,), jnp.float32),
        pltpu.VMEM((CKT, EC), jnp.int32),
        pltpu.VMEM_SHARED((NS, N_ACC), jnp.float32),
        pltpu.VMEM((STRIPE,), jnp.float32),
        pltpu.VMEM((STRIPE,), jnp.float32),
    ],
)
def _deg_kernel(dst_hbm, out_hbm, hist, didx, shared, tmp, dacc):
    cid = lax.axis_index("c")
    sid = lax.axis_index("s")
    cbase = cid * CK0
    ckm = jnp.where(cid == 0, CK0, CK1)
    zeros16 = jnp.zeros((16,), jnp.float32)
    ones16 = jnp.ones((16,), jnp.float32)

    def _zero_hist(i, _):
        hist[pl.ds(i * 16, 16)] = zeros16
        return 0

    lax.fori_loop(0, N_ACC // 16, _zero_hist, 0)

    pltpu.sync_copy(dst_hbm.at[sid], didx)

    def _count(c, _):
        for j in range(EC // 16):
            idx = didx[cbase + c, pl.ds(j * 16, 16)]
            plsc.addupdate_scatter(hist, [idx], ones16)
        return 0

    lax.fori_loop(0, ckm, _count, 0)

    pltpu.sync_copy(hist, shared.at[sid])
    plsc.subcore_barrier()

    base = sid * STRIPE

    def _zero_acc(i, _):
        dacc[pl.ds(i * 16, 16)] = zeros16
        return 0

    lax.fori_loop(0, STRIPE // 16, _zero_acc, 0)

    for k in range(NS):
        pltpu.sync_copy(shared.at[k, pl.ds(base, STRIPE)], tmp)

        def _accum(i, _):
            s = pl.ds(i * 16, 16)
            dacc[s] = dacc[s] + tmp[s]
            return 0

        lax.fori_loop(0, STRIPE // 16, _accum, 0)

    pltpu.sync_copy(dacc, out_hbm.at[cid, pl.ds(base, STRIPE)])


@functools.partial(
    pl.kernel,
    out_type=jax.ShapeDtypeStruct((NC, N_ACC, H), jnp.float32),
    mesh=_mesh,
    compiler_params=pltpu.CompilerParams(needs_layout_passes=False),
    scratch_types=[
        pltpu.VMEM_SHARED((N_ACC, H), jnp.float32),
        pltpu.VMEM((NI, EC), jnp.int32),
        pltpu.VMEM((NI, EC), jnp.int32),
        pltpu.VMEM((NBUF, EC, H), jnp.float32),
        pltpu.SemaphoreType.DMA,
        pltpu.SemaphoreType.DMA,
        pltpu.SemaphoreType.DMA,
    ],
)
def _agg_kernel(hws_hbm, src_hbm, dst_hbm, zrows_hbm, out_hbm,
                acc, sidxr, didxr, rowsr, gsem, ssem, isem):
    cid = lax.axis_index("c")
    sid = lax.axis_index("s")
    cbase = cid * CK0
    ckm = jnp.where(cid == 0, CK0, CK1)
    row0 = sid * STRIPE

    pltpu.sync_copy(zrows_hbm, acc.at[pl.ds(row0, STRIPE)])
    for j in range(NBUF - 1):
        pltpu.sync_copy(src_hbm.at[sid, cbase + j], sidxr.at[j])
        pltpu.sync_copy(dst_hbm.at[sid, cbase + j], didxr.at[j])
    for j in range(NBUF - 1, LAI):
        pltpu.async_copy(src_hbm.at[sid, cbase + j], sidxr.at[j], isem)
        pltpu.async_copy(dst_hbm.at[sid, cbase + j], didxr.at[j], isem)
    plsc.subcore_barrier()

    for j in range(NBUF - 1):
        pltpu.async_copy(hws_hbm.at[sidxr.at[j]], rowsr.at[j], gsem)

    def _edges(c, _):
        b = c % NBUF
        s_c = c % NI
        pltpu.make_async_copy(
            hws_hbm.at[sidxr.at[s_c]], rowsr.at[b], gsem).wait()
        pltpu.async_copy(rowsr.at[b], acc.at[didxr.at[s_c]], ssem, add=True)

        @pl.when(c > 0)
        def _wait_prev_scatter():
            bp = (c + NBUF - 1) % NBUF
            sp = (c + NI - 1) % NI
            pltpu.make_async_copy(
                rowsr.at[bp], acc.at[didxr.at[sp]], ssem).wait()

        @pl.when(c + NBUF - 1 < ckm)
        def _start_next_gather():
            bn = (c + NBUF - 1) % NBUF
            sn = (c + NBUF - 1) % NI
            pltpu.make_async_copy(
                src_hbm.at[sid, cbase + c + NBUF - 1], sidxr.at[sn],
                isem).wait()
            pltpu.make_async_copy(
                dst_hbm.at[sid, cbase + c + NBUF - 1], didxr.at[sn],
                isem).wait()
            pltpu.async_copy(hws_hbm.at[sidxr.at[sn]], rowsr.at[bn], gsem)

        @pl.when(c + LAI < ckm)
        def _start_next_idx_load():
            si = (c + LAI) % NI
            pltpu.async_copy(
                src_hbm.at[sid, cbase + c + LAI], sidxr.at[si], isem)
            pltpu.async_copy(
                dst_hbm.at[sid, cbase + c + LAI], didxr.at[si], isem)

        return 0

    lax.fori_loop(0, ckm, _edges, 0)
    pltpu.make_async_copy(
        rowsr.at[(ckm - 1) % NBUF],
        acc.at[didxr.at[(ckm - 1) % NI]], ssem).wait()

    plsc.subcore_barrier()
    pltpu.sync_copy(acc.at[pl.ds(row0, STRIPE)],
                    out_hbm.at[cid, pl.ds(row0, STRIPE)])


def _mm_body(x_ref, w_ref, o_ref):
    o_ref[...] = jnp.dot(
        x_ref[...], w_ref[...], preferred_element_type=jnp.float32)


def _matmul(x, w):
    grid = 10
    rb = N // grid
    return pl.pallas_call(
        _mm_body,
        grid=(grid,),
        in_specs=[
            pl.BlockSpec((rb, D), lambda i: (i, 0)),
            pl.BlockSpec((D, H), lambda i: (0, 0)),
        ],
        out_specs=pl.BlockSpec((rb, H), lambda i: (i, 0)),
        out_shape=jax.ShapeDtypeStruct((N, H), jnp.float32),
    )(x, w)


def _bn_relu(t, g_ref, be_ref):
    mean = jnp.mean(t, axis=0, keepdims=True)
    var = jnp.mean((t - mean) ** 2, axis=0, keepdims=True)
    hn = (t - mean) * lax.rsqrt(var + 1e-5) * g_ref[...] + be_ref[...]
    return jnp.maximum(hn, 0.0)


def _post1_body(p_ref, hws_ref, u_ref, b_ref, g_ref, be_ref, w2_ref,
                h1_ref, hws2_ref):
    s = p_ref[0, :N, :] + p_ref[1, :N, :] + hws_ref[...]
    t = u_ref[...] * s + b_ref[...]
    h1 = _bn_relu(t, g_ref, be_ref)
    h1_ref[...] = h1
    hws2_ref[...] = u_ref[...] * jnp.dot(
        h1, w2_ref[...], preferred_element_type=jnp.float32)


def _post2_body(p_ref, hws_ref, u_ref, b_ref, g_ref, be_ref, h2_ref):
    s = p_ref[0, :N, :] + p_ref[1, :N, :] + hws_ref[...]
    t = u_ref[...] * s + b_ref[...]
    h2_ref[...] = _bn_relu(t, g_ref, be_ref)


def kernel(x, edge_index, W1, b1, gamma1, beta1, W2, b2, gamma2, beta2):
    pad = NS * CKT * EC - E
    src_p = jnp.concatenate([edge_index[0], jnp.zeros((pad,), jnp.int32)])
    dst_p = jnp.concatenate([edge_index[1], jnp.full((pad,), N, jnp.int32)])
    src3 = src_p.reshape(NS, CKT, EC)
    dst3 = dst_p.reshape(NS, CKT, EC)

    deg2 = _deg_kernel(dst3)
    deg = deg2[0, :N] + deg2[1, :N] + 1.0
    u_col = lax.rsqrt(deg)[:, None]

    zrows = jnp.zeros((STRIPE, H), jnp.float32)
    b1r, g1r, be1r = b1[None, :], gamma1[None, :], beta1[None, :]
    b2r, g2r, be2r = b2[None, :], gamma2[None, :], beta2[None, :]

    hws1 = u_col * _matmul(x, W1)
    p1 = _agg_kernel(hws1, src3, dst3, zrows)

    h1, hws2 = pl.pallas_call(
        _post1_body,
        out_shape=(jax.ShapeDtypeStruct((N, H), jnp.float32),
                   jax.ShapeDtypeStruct((N, H), jnp.float32)),
    )(p1, hws1, u_col, b1r, g1r, be1r, W2)

    p2 = _agg_kernel(hws2, src3, dst3, zrows)

    h2 = pl.pallas_call(
        _post2_body,
        out_shape=jax.ShapeDtypeStruct((N, H), jnp.float32),
    )(p2, hws2, u_col, b2r, g2r, be2r)

    return (h1, h2)


# speedup vs baseline: 1.0212x; 1.0009x over previous
import functools

import jax
import jax.numpy as jnp
from jax import lax
from jax.experimental import pallas as pl
from jax.experimental.pallas import tpu as pltpu
from jax.experimental.pallas import tpu_sc as plsc

N = 10000
E = 320000
D = 128
H = 128

NC = 2
NS = 16
EC = 64
CKT = -(-E // (NS * EC))
CK0 = 196
CK1 = CKT - CK0
NBUF = 5
NI = 8
LAI = 6
N_ACC = 10240
STRIPE = N_ACC // NS

_mesh = plsc.VectorSubcoreMesh(core_axis_name="c", subcore_axis_name="s")


@functools.partial(
    pl.kernel,
    out_type=jax.ShapeDtypeStruct((NC, N_ACC), jnp.float32),
    mesh=_mesh,
    compiler_params=pltpu.CompilerParams(needs_layout_passes=False),
    scratch_types=[
        pltpu.VMEM((N_ACC,), jnp.float32),
        pltpu.VMEM((CKT, EC), jnp.int32),
        pltpu.VMEM_SHARED((NS, N_ACC), jnp.float32),
        pltpu.VMEM((STRIPE,), jnp.float32),
        pltpu.VMEM((STRIPE,), jnp.float32),
    ],
)
def _deg_kernel(dst_hbm, out_hbm, hist, didx, shared, tmp, dacc):
    cid = lax.axis_index("c")
    sid = lax.axis_index("s")
    cbase = cid * CK0
    ckm = jnp.where(cid == 0, CK0, CK1)
    zeros16 = jnp.zeros((16,), jnp.float32)
    ones16 = jnp.ones((16,), jnp.float32)

    def _zero_hist(i, _):
        hist[pl.ds(i * 16, 16)] = zeros16
        return 0

    lax.fori_loop(0, N_ACC // 16, _zero_hist, 0)

    pltpu.sync_copy(dst_hbm.at[sid], didx)

    def _count(c, _):
        for j in range(EC // 16):
            idx = didx[cbase + c, pl.ds(j * 16, 16)]
            plsc.addupdate_scatter(hist, [idx], ones16)
        return 0

    lax.fori_loop(0, ckm, _count, 0)

    pltpu.sync_copy(hist, shared.at[sid])
    plsc.subcore_barrier()

    base = sid * STRIPE

    def _zero_acc(i, _):
        dacc[pl.ds(i * 16, 16)] = zeros16
        return 0

    lax.fori_loop(0, STRIPE // 16, _zero_acc, 0)

    for k in range(NS):
        pltpu.sync_copy(shared.at[k, pl.ds(base, STRIPE)], tmp)

        def _accum(i, _):
            s = pl.ds(i * 16, 16)
            dacc[s] = dacc[s] + tmp[s]
            return 0

        lax.fori_loop(0, STRIPE // 16, _accum, 0)

    pltpu.sync_copy(dacc, out_hbm.at[cid, pl.ds(base, STRIPE)])


@functools.partial(
    pl.kernel,
    out_type=jax.ShapeDtypeStruct((NC, N_ACC, H), jnp.float32),
    mesh=_mesh,
    compiler_params=pltpu.CompilerParams(needs_layout_passes=False),
    scratch_types=[
        pltpu.VMEM_SHARED((N_ACC, H), jnp.float32),
        pltpu.VMEM((NI, EC), jnp.int32),
        pltpu.VMEM((NI, EC), jnp.int32),
        pltpu.VMEM((NBUF, EC, H), jnp.float32),
        pltpu.SemaphoreType.DMA,
        pltpu.SemaphoreType.DMA,
        pltpu.SemaphoreType.DMA,
    ],
)
def _agg_kernel(hws_hbm, src_hbm, dst_hbm, zrows_hbm, out_hbm,
                acc, sidxr, didxr, rowsr, gsem, ssem, isem):
    cid = lax.axis_index("c")
    sid = lax.axis_index("s")
    cbase = cid * CK0
    ckm = jnp.where(cid == 0, CK0, CK1)
    row0 = sid * STRIPE

    pltpu.sync_copy(zrows_hbm, acc.at[pl.ds(row0, STRIPE)])
    for j in range(NBUF - 1):
        pltpu.sync_copy(src_hbm.at[sid, cbase + j], sidxr.at[j])
        pltpu.sync_copy(dst_hbm.at[sid, cbase + j], didxr.at[j])
    for j in range(NBUF - 1, LAI):
        pltpu.async_copy(src_hbm.at[sid, cbase + j], sidxr.at[j], isem)
        pltpu.async_copy(dst_hbm.at[sid, cbase + j], didxr.at[j], isem)
    plsc.subcore_barrier()

    for j in range(NBUF - 1):
        pltpu.async_copy(hws_hbm.at[sidxr.at[j]], rowsr.at[j], gsem)

    def _edges(c, _):
        b = c % NBUF
        s_c = c % NI
        pltpu.make_async_copy(
            hws_hbm.at[sidxr.at[s_c]], rowsr.at[b], gsem).wait()
        pltpu.async_copy(rowsr.at[b], acc.at[didxr.at[s_c]], ssem, add=True)

        @pl.when(c > 0)
        def _wait_prev_scatter():
            bp = (c + NBUF - 1) % NBUF
            sp = (c + NI - 1) % NI
            pltpu.make_async_copy(
                rowsr.at[bp], acc.at[didxr.at[sp]], ssem).wait()

        @pl.when(c + NBUF - 1 < ckm)
        def _start_next_gather():
            bn = (c + NBUF - 1) % NBUF
            sn = (c + NBUF - 1) % NI
            pltpu.make_async_copy(
                src_hbm.at[sid, cbase + c + NBUF - 1], sidxr.at[sn],
                isem).wait()
            pltpu.make_async_copy(
                dst_hbm.at[sid, cbase + c + NBUF - 1], didxr.at[sn],
                isem).wait()
            pltpu.async_copy(hws_hbm.at[sidxr.at[sn]], rowsr.at[bn], gsem)

        @pl.when(c + LAI < ckm)
        def _start_next_idx_load():
            si = (c + LAI) % NI
            pltpu.async_copy(
                src_hbm.at[sid, cbase + c + LAI], sidxr.at[si], isem)
            pltpu.async_copy(
                dst_hbm.at[sid, cbase + c + LAI], didxr.at[si], isem)

        return 0

    lax.fori_loop(0, ckm, _edges, 0)
    pltpu.make_async_copy(
        rowsr.at[(ckm - 1) % NBUF],
        acc.at[didxr.at[(ckm - 1) % NI]], ssem).wait()

    plsc.subcore_barrier()
    pltpu.sync_copy(acc.at[pl.ds(row0, STRIPE)],
                    out_hbm.at[cid, pl.ds(row0, STRIPE)])


def _mm_body(x_ref, w_ref, o_ref):
    o_ref[...] = jnp.dot(
        x_ref[...], w_ref[...], preferred_element_type=jnp.float32)


def _matmul(x, w):
    grid = 10
    rb = N // grid
    return pl.pallas_call(
        _mm_body,
        grid=(grid,),
        in_specs=[
            pl.BlockSpec((rb, D), lambda i: (i, 0)),
            pl.BlockSpec((D, H), lambda i: (0, 0)),
        ],
        out_specs=pl.BlockSpec((rb, H), lambda i: (i, 0)),
        out_shape=jax.ShapeDtypeStruct((N, H), jnp.float32),
    )(x, w)


def _bn_relu(t, g_ref, be_ref):
    mean = jnp.mean(t, axis=0, keepdims=True)
    var = jnp.mean((t - mean) ** 2, axis=0, keepdims=True)
    hn = (t - mean) * lax.rsqrt(var + 1e-5) * g_ref[...] + be_ref[...]
    return jnp.maximum(hn, 0.0)


def _post1_body(p_ref, hws_ref, u_ref, b_ref, g_ref, be_ref, w2_ref,
                h1_ref, hws2_ref):
    s = p_ref[0, :N, :] + p_ref[1, :N, :] + hws_ref[...]
    t = u_ref[...] * s + b_ref[...]
    h1 = _bn_relu(t, g_ref, be_ref)
    h1_ref[...] = h1
    hws2_ref[...] = u_ref[...] * jnp.dot(
        h1, w2_ref[...], preferred_element_type=jnp.float32)


def _post2_body(p_ref, hws_ref, u_ref, b_ref, g_ref, be_ref, h2_ref):
    s = p_ref[0, :N, :] + p_ref[1, :N, :] + hws_ref[...]
    t = u_ref[...] * s + b_ref[...]
    h2_ref[...] = _bn_relu(t, g_ref, be_ref)


def kernel(x, edge_index, W1, b1, gamma1, beta1, W2, b2, gamma2, beta2):
    pad = NS * CKT * EC - E
    src_p = jnp.concatenate([edge_index[0], jnp.zeros((pad,), jnp.int32)])
    dst_p = jnp.concatenate([edge_index[1], jnp.full((pad,), N, jnp.int32)])
    src3 = src_p.reshape(NS, CKT, EC)
    dst3 = dst_p.reshape(NS, CKT, EC)

    deg2 = _deg_kernel(dst3)
    deg = deg2[0, :N] + deg2[1, :N] + 1.0
    u_col = lax.rsqrt(deg)[:, None]

    zrows = jnp.zeros((STRIPE, H), jnp.float32)
    b1r, g1r, be1r = b1[None, :], gamma1[None, :], beta1[None, :]
    b2r, g2r, be2r = b2[None, :], gamma2[None, :], beta2[None, :]

    hws1 = u_col * _matmul(x, W1)
    p1 = _agg_kernel(hws1, src3, dst3, zrows)

    h1, hws2 = pl.pallas_call(
        _post1_body,
        out_shape=(jax.ShapeDtypeStruct((N, H), jnp.float32),
                   jax.ShapeDtypeStruct((N, H), jnp.float32)),
    )(p1, hws1, u_col, b1r, g1r, be1r, W2)

    p2 = _agg_kernel(hws2, src3, dst3, zrows)

    h2 = pl.pallas_call(
        _post2_body,
        out_shape=jax.ShapeDtypeStruct((N, H), jnp.float32),
    )(p2, hws2, u_col, b2r, g2r, be2r)

    return (h1, h2)
